# baseline (speedup 1.0000x reference)
import functools

import jax
import jax.numpy as jnp
import numpy as np
from jax import lax
from jax.experimental import pallas as pl
from jax.experimental.pallas import tpu as pltpu
from jax.experimental.pallas import tpu_sc as plsc

RC = 5.0

_NC = 2
_NSC = 16
_NW = _NC * _NSC
_CH = 128

_BE = 2000
_BN = 2000


def _mesh():
    return plsc.VectorSubcoreMesh(
        core_axis_name="c", subcore_axis_name="s",
        num_cores=_NC, num_subcores=_NSC)


_CHV = 1280


def _sc_edge_vec(px, py, pz, src2d, dst2d):
    n = px.shape[0]
    nchunk_pad = src2d.shape[0]
    per = nchunk_pad // _NW
    epad = nchunk_pad * _CHV

    @functools.partial(
        pl.kernel,
        out_type=tuple(jax.ShapeDtypeStruct((epad,), jnp.float32)
                       for _ in range(3)),
        mesh=_mesh(),
        scratch_types=[
            pltpu.VMEM((n,), jnp.float32),
            pltpu.VMEM((n,), jnp.float32),
            pltpu.VMEM((n,), jnp.float32),
            pltpu.VMEM((per, _CHV), jnp.int32),
            pltpu.VMEM((per, _CHV), jnp.int32),
            pltpu.VMEM((_CHV,), jnp.float32),
            pltpu.VMEM((_CHV,), jnp.float32),
            pltpu.VMEM((_CHV,), jnp.float32),
        ],
        compiler_params=pltpu.CompilerParams(needs_layout_passes=False),
    )
    def k(px_hbm, py_hbm, pz_hbm, src_hbm, dst_hbm, vx_hbm, vy_hbm, vz_hbm,
          px_v, py_v, pz_v, si_v, di_v, vx_v, vy_v, vz_v):
        wid = lax.axis_index("s") * _NC + lax.axis_index("c")
        c0 = wid * per
        pltpu.sync_copy(src_hbm.at[pl.ds(c0, per)], si_v)
        pltpu.sync_copy(dst_hbm.at[pl.ds(c0, per)], di_v)
        pltpu.sync_copy(px_hbm, px_v)
        pltpu.sync_copy(py_hbm, py_v)
        pltpu.sync_copy(pz_hbm, pz_v)

        for t in range(per):
            def jstep(j, c):
                o = j * 16
                s16 = si_v[t, pl.ds(o, 16)]
                d16 = di_v[t, pl.ds(o, 16)]
                vx_v[pl.ds(o, 16)] = (plsc.load_gather(px_v, [d16]) -
                                      plsc.load_gather(px_v, [s16]))
                vy_v[pl.ds(o, 16)] = (plsc.load_gather(py_v, [d16]) -
                                      plsc.load_gather(py_v, [s16]))
                vz_v[pl.ds(o, 16)] = (plsc.load_gather(pz_v, [d16]) -
                                      plsc.load_gather(pz_v, [s16]))
                return c

            lax.fori_loop(0, _CHV // 16, jstep, 0)
            base = (c0 + t) * _CHV
            pltpu.sync_copy(vx_v, vx_hbm.at[pl.ds(base, _CHV)])
            pltpu.sync_copy(vy_v, vy_hbm.at[pl.ds(base, _CHV)])
            pltpu.sync_copy(vz_v, vz_hbm.at[pl.ds(base, _CHV)])

        return None

    return k(px, py, pz, src2d, dst2d)


def _sc_gather_rows(table, idx2d):
    nchunk_pad = idx2d.shape[0]
    d = table.shape[1]
    per = nchunk_pad // _NW
    kk = 3
    ngroup = -(-per // kk)

    @functools.partial(
        pl.kernel,
        out_type=jax.ShapeDtypeStruct((nchunk_pad * _CH, d), jnp.float32),
        mesh=_mesh(),
        scratch_types=[
            pltpu.VMEM((per, _CH), jnp.int32),
            pltpu.VMEM((2, kk, _CH, d), jnp.float32),
            pltpu.SemaphoreType.DMA,
            pltpu.SemaphoreType.DMA,
            pltpu.SemaphoreType.DMA,
            pltpu.SemaphoreType.DMA,
        ],
    )
    def k(tab_hbm, idx_hbm, out_hbm, idx_v, rows_v, sg0, sg1, sw0, sw1):
        wid = lax.axis_index("s") * _NC + lax.axis_index("c")
        c0 = wid * per
        pltpu.sync_copy(idx_hbm.at[pl.ds(c0, per)], idx_v)
        sg = (sg0, sg1)
        sw = (sw0, sw1)
        gat = {}
        wbk = {}
        for g in range(ngroup):
            bank = g % 2
            if g >= 2:
                for dsc in wbk.pop(g - 2):
                    dsc.wait()
            fired = []
            for j in range(kk):
                t = g * kk + j
                if t >= per:
                    break
                dsc = pltpu.async_copy(
                    tab_hbm.at[idx_v.at[t]], rows_v.at[bank, j], sg[bank])
                fired.append((t, dsc))
            gat[g] = fired
            if g >= 1:
                pb = (g - 1) % 2
                wfired = []
                for (t, dsc) in gat.pop(g - 1):
                    dsc.wait()
                    wdsc = pltpu.async_copy(
                        rows_v.at[pb, t - (g - 1) * kk],
                        out_hbm.at[pl.ds((c0 + t) * _CH, _CH)], sw[pb])
                    wfired.append(wdsc)
                wbk[g - 1] = wfired
        g = ngroup
        pb = (g - 1) % 2
        wfired = []
        for (t, dsc) in gat.pop(g - 1, []):
            dsc.wait()
            wdsc = pltpu.async_copy(
                rows_v.at[pb, t - (g - 1) * kk],
                out_hbm.at[pl.ds((c0 + t) * _CH, _CH)], sw[pb])
            wfired.append(wdsc)
        wbk[g - 1] = wfired
        for fired in wbk.values():
            for dsc in fired:
                dsc.wait()

    return k(table, idx2d)


def _sc_scatter_add(msg, dst2d, npad):
    e, d = msg.shape
    nchunk_pad = dst2d.shape[0]
    per = nchunk_pad // _NW
    rows_tile = npad // _NSC
    stage = 32
    nstage = rows_tile // stage

    @functools.partial(
        pl.kernel,
        out_type=jax.ShapeDtypeStruct((_NC, npad, d), jnp.float32),
        mesh=_mesh(),
        scratch_types=[
            pltpu.VMEM((per, _CH), jnp.int32),
            pltpu.VMEM((2, _CH, d), jnp.float32),
            pltpu.VMEM((stage, d), jnp.float32),
            pltpu.VMEM_SHARED((npad, d), jnp.float32),
            pltpu.SemaphoreType.DMA,
            pltpu.SemaphoreType.DMA,
            pltpu.SemaphoreType.DMA,
            pltpu.SemaphoreType.DMA,
        ],
    )
    def k(msg_hbm, idx_hbm, out_hbm, idx_v, rows_v, stage_v, acc_sh,
          sm0, sm1, ss0, ss1):
        cid = lax.axis_index("c")
        sid = lax.axis_index("s")
        wid = sid * _NC + cid
        c0 = wid * per
        ipre = pltpu.async_copy(idx_hbm.at[pl.ds(c0, per)], idx_v, sm1)

        def zbody(i, carry):
            for q in range(d // 16):
                stage_v[i, pl.ds(q * 16, 16)] = jnp.zeros((16,), jnp.float32)
            return carry

        lax.fori_loop(0, stage, zbody, 0)

        def zcopy(j, carry):
            pltpu.sync_copy(
                stage_v, acc_sh.at[pl.ds(sid * rows_tile + j * stage, stage)])
            return carry

        lax.fori_loop(0, nstage, zcopy, 0)
        ipre.wait()
        plsc.subcore_barrier()

        sm = (sm0, sm1)
        ss = (ss0, ss1)
        loads = {}
        adds = {}
        for t in range(per):
            b = t % 2
            if t >= 2:
                adds.pop(t - 2).wait()
            base = jnp.minimum((c0 + t) * _CH, e - _CH)
            loads[t] = pltpu.async_copy(
                msg_hbm.at[pl.ds(base, _CH)], rows_v.at[b], sm[b])
            if t >= 1:
                loads.pop(t - 1).wait()
                adds[t - 1] = pltpu.async_copy(
                    rows_v.at[(t - 1) % 2], acc_sh.at[idx_v.at[t - 1]],
                    ss[(t - 1) % 2], add=True)
        loads.pop(per - 1).wait()
        adds[per - 1] = pltpu.async_copy(
            rows_v.at[(per - 1) % 2], acc_sh.at[idx_v.at[per - 1]],
            ss[(per - 1) % 2], add=True)
        for t in list(adds):
            adds.pop(t).wait()
        plsc.subcore_barrier()

        def ocopy(j, carry):
            r0 = sid * rows_tile + j * stage
            pltpu.sync_copy(acc_sh.at[pl.ds(r0, stage)], stage_v)
            pltpu.sync_copy(stage_v, out_hbm.at[cid, pl.ds(r0, stage)])
            return carry

        lax.fori_loop(0, nstage, ocopy, 0)

    return k(msg, dst2d)


def _silu(t):
    return t / (1.0 + jnp.exp(-t))


def _geom_body(vx_ref, vy_ref, vz_ref, r_ref, sh_ref):
    x = vx_ref[:, :]
    y = vy_ref[:, :]
    z = vz_ref[:, :]
    r = jnp.sqrt(x * x + y * y + z * z) + 1e-6
    r_ref[:, :] = r
    inv = 1.0 / r
    x = x * inv
    y = y * inv
    z = z * inv
    xx = x * x
    yy = y * y
    zz = z * z
    one = jnp.ones_like(x)
    cols = [
        one,
        x, y, z,
        x * y, y * z, 2.0 * zz - xx - yy, z * x, xx - yy,
        y * (3.0 * xx - yy), x * y * z, y * (4.0 * zz - xx - yy),
        z * (2.0 * zz - 3.0 * xx - 3.0 * yy), x * (4.0 * zz - xx - yy),
        z * (xx - yy), x * (xx - 3.0 * yy),
    ]
    sh_ref[:, :] = jnp.concatenate(cols, axis=1)


def _tc_geom(vx, vy, vz, e):
    grid = e // _BE
    return pl.pallas_call(
        _geom_body,
        grid=(grid,),
        in_specs=[
            pl.BlockSpec((_BE, 1), lambda i: (i, 0)),
            pl.BlockSpec((_BE, 1), lambda i: (i, 0)),
            pl.BlockSpec((_BE, 1), lambda i: (i, 0)),
        ],
        out_specs=[
            pl.BlockSpec((_BE, 1), lambda i: (i, 0)),
            pl.BlockSpec((_BE, 16), lambda i: (i, 0)),
        ],
        out_shape=[
            jax.ShapeDtypeStruct((e, 1), jnp.float32),
            jax.ShapeDtypeStruct((e, 16), jnp.float32),
        ],
    )(vx.reshape(-1, 1), vy.reshape(-1, 1), vz.reshape(-1, 1))


def _init_body(ns, at_ref, ne_ref, wmsg_ref, x_ref, y_ref):
    at = at_ref[:, :]
    x = jnp.zeros((at.shape[0], ne_ref.shape[1]), jnp.float32)
    for s in range(ns):
        x = jnp.where(at == s, ne_ref[s:s + 1, :], x)
    x_ref[:, :] = x
    y_ref[:, :] = jnp.dot(x, wmsg_ref[:, :], preferred_element_type=jnp.float32)


def _tc_init(atom_types2d, node_embed, wmsg0):
    n = atom_types2d.shape[0]
    ns, d = node_embed.shape
    grid = n // _BN
    return pl.pallas_call(
        functools.partial(_init_body, ns),
        grid=(grid,),
        in_specs=[
            pl.BlockSpec((_BN, 1), lambda i: (i, 0)),
            pl.BlockSpec((ns, d), lambda i: (0, 0)),
            pl.BlockSpec((d, d), lambda i: (0, 0)),
        ],
        out_specs=[
            pl.BlockSpec((_BN, d), lambda i: (i, 0)),
            pl.BlockSpec((_BN, d), lambda i: (i, 0)),
        ],
        out_shape=[
            jax.ShapeDtypeStruct((n, d), jnp.float32),
            jax.ShapeDtypeStruct((n, d), jnp.float32),
        ],
    )(atom_types2d, node_embed, wmsg0)


def _bessel_block(r, nb):
    n = lax.broadcasted_iota(jnp.int32, (1, nb), 1).astype(jnp.float32) + 1.0
    s = jnp.sin(r * (np.pi / RC) * n)
    rb = s * (np.sqrt(2.0 / RC) / r)
    u = jnp.clip(r * (1.0 / RC), 0.0, 1.0)
    u2 = u * u
    u6 = u2 * u2 * u2
    fc = 1.0 - 28.0 * u6 + 48.0 * u6 * u - 21.0 * u6 * u2
    return rb * fc


def _edge_body(first, nb, r_ref, sh_ref, g_ref, wrad_ref, wsh_ref, wedge_ref,
               *rest):
    if first:
        (wei_ref, msg_ref, eo_ref) = rest
    else:
        (e_ref, msg_ref, eo_ref) = rest
    r = r_ref[:, :]
    rb = _bessel_block(r, nb)
    radial_w = jnp.dot(rb, wrad_ref[:, :], preferred_element_type=jnp.float32)
    sh_w = jnp.dot(sh_ref[:, :], wsh_ref[:, :],
                   preferred_element_type=jnp.float32)
    if first:
        e = jnp.dot(rb, wei_ref[:, :], preferred_element_type=jnp.float32)
    else:
        e = e_ref[:, :]
    msg = g_ref[:, :] * radial_w * sh_w + e
    msg_ref[:, :] = msg
    t = jnp.dot(msg, wedge_ref[:, :], preferred_element_type=jnp.float32)
    eo_ref[:, :] = e + _silu(t)


def _tc_edge(r, sh, g, wrad, wsh, wedge, e_or_wei, first):
    e_rows = r.shape[0]
    d = g.shape[1]
    nb = wrad.shape[0]
    grid = e_rows // _BE
    full = lambda a, b: pl.BlockSpec((a, b), lambda i: (0, 0))
    in_specs = [
        pl.BlockSpec((_BE, 1), lambda i: (i, 0)),
        pl.BlockSpec((_BE, 16), lambda i: (i, 0)),
        pl.BlockSpec((_BE, d), lambda i: (i, 0)),
        full(nb, d),
        full(16, d),
        full(d, d),
    ]
    if first:
        in_specs.append(full(nb, d))
    else:
        in_specs.append(pl.BlockSpec((_BE, d), lambda i: (i, 0)))
    return pl.pallas_call(
        functools.partial(_edge_body, first, nb),
        grid=(grid,),
        in_specs=in_specs,
        out_specs=[
            pl.BlockSpec((_BE, d), lambda i: (i, 0)),
            pl.BlockSpec((_BE, d), lambda i: (i, 0)),
        ],
        out_shape=[
            jax.ShapeDtypeStruct((e_rows, d), jnp.float32),
            jax.ShapeDtypeStruct((e_rows, d), jnp.float32),
        ],
    )(r, sh, g, wrad, wsh, wedge, e_or_wei)


def _upd_body(last, aggp_ref, x_ref, wupd_ref, wmsg_ref, xo_ref, y_ref):
    agg = aggp_ref[0] + aggp_ref[1]
    t = jnp.dot(agg, wupd_ref[:, :], preferred_element_type=jnp.float32)
    xo = x_ref[:, :] + _silu(t)
    xo_ref[:, :] = xo
    if not last:
        y_ref[:, :] = jnp.dot(xo, wmsg_ref[:, :],
                              preferred_element_type=jnp.float32)


def _tc_update(aggp, x, wupd, wmsg_next, last):
    n, d = x.shape
    grid = n // _BN
    return pl.pallas_call(
        functools.partial(_upd_body, last),
        grid=(grid,),
        in_specs=[
            pl.BlockSpec((_NC, _BN, d), lambda i: (0, i, 0)),
            pl.BlockSpec((_BN, d), lambda i: (i, 0)),
            pl.BlockSpec((d, d), lambda i: (0, 0)),
            pl.BlockSpec((d, d), lambda i: (0, 0)),
        ],
        out_specs=[
            pl.BlockSpec((_BN, d), lambda i: (i, 0)),
            pl.BlockSpec((_BN, d), lambda i: (i, 0)),
        ],
        out_shape=[
            jax.ShapeDtypeStruct((n, d), jnp.float32),
            jax.ShapeDtypeStruct((n, d), jnp.float32),
        ],
    )(aggp, x, wupd, wmsg_next)


def kernel(pos, edge_index, atom_types, node_embed, W_edge_init, W_rad,
           W_sh, W_msg, W_upd, W_edge):
    n, _ = pos.shape
    e = edge_index.shape[1]
    nl = W_rad.shape[0]
    d = node_embed.shape[1]
    npad = -(-n // 1280) * 1280

    src = edge_index[0]
    dst = edge_index[1]
    px = jnp.asarray(pos[:, 0])
    py = jnp.asarray(pos[:, 1])
    pz = jnp.asarray(pos[:, 2])
    at2d = atom_types.reshape(n, 1)

    nchunk = e // _CH
    ncp = -(-nchunk // _NW) * _NW
    src2d = jnp.pad(src.reshape(nchunk, _CH), ((0, ncp - nchunk), (0, 0)))
    dst2d = jnp.pad(dst.reshape(nchunk, _CH), ((0, ncp - nchunk), (0, 0)),
                    constant_values=npad - 1)
    nchv = e // _CHV
    ncpv = -(-nchv // _NW) * _NW
    src2v = jnp.pad(src.reshape(nchv, _CHV), ((0, ncpv - nchv), (0, 0)))
    dst2v = jnp.pad(dst.reshape(nchv, _CHV), ((0, ncpv - nchv), (0, 0)))

    vx, vy, vz = _sc_edge_vec(px, py, pz, src2v, dst2v)
    r, sh = _tc_geom(vx, vy, vz, e)
    x, y = _tc_init(at2d, node_embed, W_msg[0])

    e_cur = None
    for l in range(nl):
        g = _sc_gather_rows(y, src2d)
        if l == 0:
            msg, e_cur = _tc_edge(r, sh, g, W_rad[0], W_sh[0], W_edge[0],
                                  W_edge_init, first=True)
        else:
            msg, e_cur = _tc_edge(r, sh, g, W_rad[l], W_sh[l], W_edge[l],
                                  e_cur, first=False)
        aggp = _sc_scatter_add(msg, dst2d, npad)
        wmsg_next = W_msg[l + 1] if l + 1 < nl else W_msg[l]
        x, y = _tc_update(aggp, x, W_upd[l], wmsg_next,
                          last=(l + 1 == nl))
    return (x, e_cur)


# speedup vs baseline: 1.1806x; 1.1806x over previous
import functools

import jax
import jax.numpy as jnp
import numpy as np
from jax import lax
from jax.experimental import pallas as pl
from jax.experimental.pallas import tpu as pltpu
from jax.experimental.pallas import tpu_sc as plsc

RC = 5.0

_NC = 2
_NSC = 16
_NW = _NC * _NSC
_CH = 128

_BE = 2048
_BN = 2000


def _mesh():
    return plsc.VectorSubcoreMesh(
        core_axis_name="c", subcore_axis_name="s",
        num_cores=_NC, num_subcores=_NSC)


_CHV = 1024


def _sc_edge_vec(px, py, pz, srcp, dstp):
    n = px.shape[0]
    epad = srcp.shape[0]
    per = epad // _NW

    @functools.partial(
        pl.kernel,
        out_type=tuple(jax.ShapeDtypeStruct((epad,), jnp.float32)
                       for _ in range(3)),
        mesh=_mesh(),
        scratch_types=[
            pltpu.VMEM((n,), jnp.float32),
            pltpu.VMEM((n,), jnp.float32),
            pltpu.VMEM((n,), jnp.float32),
            pltpu.VMEM((per,), jnp.int32),
            pltpu.VMEM((per,), jnp.int32),
            pltpu.VMEM((per,), jnp.float32),
            pltpu.VMEM((per,), jnp.float32),
            pltpu.VMEM((per,), jnp.float32),
        ],
        compiler_params=pltpu.CompilerParams(needs_layout_passes=False),
    )
    def k(px_hbm, py_hbm, pz_hbm, src_hbm, dst_hbm, vx_hbm, vy_hbm, vz_hbm,
          px_v, py_v, pz_v, si_v, di_v, vx_v, vy_v, vz_v):
        wid = lax.axis_index("s") * _NC + lax.axis_index("c")
        c0 = pl.multiple_of(wid * per, per)
        pltpu.sync_copy(src_hbm.at[pl.ds(c0, per)], si_v)
        pltpu.sync_copy(dst_hbm.at[pl.ds(c0, per)], di_v)
        pltpu.sync_copy(px_hbm, px_v)
        pltpu.sync_copy(py_hbm, py_v)
        pltpu.sync_copy(pz_hbm, pz_v)

        def jstep(j, c):
            o = j * 16
            s16 = si_v[pl.ds(o, 16)]
            d16 = di_v[pl.ds(o, 16)]
            vx_v[pl.ds(o, 16)] = (plsc.load_gather(px_v, [d16]) -
                                  plsc.load_gather(px_v, [s16]))
            vy_v[pl.ds(o, 16)] = (plsc.load_gather(py_v, [d16]) -
                                  plsc.load_gather(py_v, [s16]))
            vz_v[pl.ds(o, 16)] = (plsc.load_gather(pz_v, [d16]) -
                                  plsc.load_gather(pz_v, [s16]))
            return c

        lax.fori_loop(0, per // 16, jstep, 0)
        pltpu.sync_copy(vx_v, vx_hbm.at[pl.ds(c0, per)])
        pltpu.sync_copy(vy_v, vy_hbm.at[pl.ds(c0, per)])
        pltpu.sync_copy(vz_v, vz_hbm.at[pl.ds(c0, per)])

    return k(px, py, pz, srcp, dstp)


def _sc_gather_rows(table, idx2d):
    nchunk_pad = idx2d.shape[0]
    d = table.shape[1]
    per = nchunk_pad // _NW
    kk = 3
    ngroup = -(-per // kk)

    @functools.partial(
        pl.kernel,
        out_type=jax.ShapeDtypeStruct((nchunk_pad * _CH, d), jnp.float32),
        mesh=_mesh(),
        scratch_types=[
            pltpu.VMEM((per, _CH), jnp.int32),
            pltpu.VMEM((2, kk, _CH, d), jnp.float32),
            pltpu.SemaphoreType.DMA,
            pltpu.SemaphoreType.DMA,
            pltpu.SemaphoreType.DMA,
            pltpu.SemaphoreType.DMA,
        ],
    )
    def k(tab_hbm, idx_hbm, out_hbm, idx_v, rows_v, sg0, sg1, sw0, sw1):
        wid = lax.axis_index("s") * _NC + lax.axis_index("c")
        c0 = wid * per
        pltpu.sync_copy(idx_hbm.at[pl.ds(c0, per)], idx_v)
        sg = (sg0, sg1)
        sw = (sw0, sw1)
        gat = {}
        wbk = {}
        for g in range(ngroup):
            bank = g % 2
            if g >= 2:
                for dsc in wbk.pop(g - 2):
                    dsc.wait()
            fired = []
            for j in range(kk):
                t = g * kk + j
                if t >= per:
                    break
                dsc = pltpu.async_copy(
                    tab_hbm.at[idx_v.at[t]], rows_v.at[bank, j], sg[bank])
                fired.append((t, dsc))
            gat[g] = fired
            if g >= 1:
                pb = (g - 1) % 2
                wfired = []
                for (t, dsc) in gat.pop(g - 1):
                    dsc.wait()
                    wdsc = pltpu.async_copy(
                        rows_v.at[pb, t - (g - 1) * kk],
                        out_hbm.at[pl.ds((c0 + t) * _CH, _CH)], sw[pb])
                    wfired.append(wdsc)
                wbk[g - 1] = wfired
        g = ngroup
        pb = (g - 1) % 2
        wfired = []
        for (t, dsc) in gat.pop(g - 1, []):
            dsc.wait()
            wdsc = pltpu.async_copy(
                rows_v.at[pb, t - (g - 1) * kk],
                out_hbm.at[pl.ds((c0 + t) * _CH, _CH)], sw[pb])
            wfired.append(wdsc)
        wbk[g - 1] = wfired
        for fired in wbk.values():
            for dsc in fired:
                dsc.wait()

    return k(table, idx2d)


def _sc_scatter_add(msg, dst2d, npad, evalid):
    e, d = msg.shape
    nchunk_pad = dst2d.shape[0]
    per = nchunk_pad // _NW
    rows_tile = npad // _NSC
    stage = 32
    nstage = rows_tile // stage

    @functools.partial(
        pl.kernel,
        out_type=jax.ShapeDtypeStruct((_NC, npad, d), jnp.float32),
        mesh=_mesh(),
        scratch_types=[
            pltpu.VMEM((per, _CH), jnp.int32),
            pltpu.VMEM((2, _CH, d), jnp.float32),
            pltpu.VMEM((stage, d), jnp.float32),
            pltpu.VMEM_SHARED((npad, d), jnp.float32),
            pltpu.SemaphoreType.DMA,
            pltpu.SemaphoreType.DMA,
            pltpu.SemaphoreType.DMA,
            pltpu.SemaphoreType.DMA,
        ],
    )
    def k(msg_hbm, idx_hbm, out_hbm, idx_v, rows_v, stage_v, acc_sh,
          sm0, sm1, ss0, ss1):
        cid = lax.axis_index("c")
        sid = lax.axis_index("s")
        wid = sid * _NC + cid
        c0 = wid * per
        ipre = pltpu.async_copy(idx_hbm.at[pl.ds(c0, per)], idx_v, sm1)

        def zbody(i, carry):
            for q in range(d // 16):
                stage_v[i, pl.ds(q * 16, 16)] = jnp.zeros((16,), jnp.float32)
            return carry

        lax.fori_loop(0, stage, zbody, 0)

        def zcopy(j, carry):
            pltpu.sync_copy(
                stage_v, acc_sh.at[pl.ds(sid * rows_tile + j * stage, stage)])
            return carry

        lax.fori_loop(0, nstage, zcopy, 0)
        ipre.wait()
        plsc.subcore_barrier()

        sm = (sm0, sm1)
        ss = (ss0, ss1)
        loads = {}
        adds = {}
        for t in range(per):
            b = t % 2
            if t >= 2:
                adds.pop(t - 2).wait()
            base = jnp.minimum((c0 + t) * _CH, evalid - _CH)
            loads[t] = pltpu.async_copy(
                msg_hbm.at[pl.ds(base, _CH)], rows_v.at[b], sm[b])
            if t >= 1:
                loads.pop(t - 1).wait()
                adds[t - 1] = pltpu.async_copy(
                    rows_v.at[(t - 1) % 2], acc_sh.at[idx_v.at[t - 1]],
                    ss[(t - 1) % 2], add=True)
        loads.pop(per - 1).wait()
        adds[per - 1] = pltpu.async_copy(
            rows_v.at[(per - 1) % 2], acc_sh.at[idx_v.at[per - 1]],
            ss[(per - 1) % 2], add=True)
        for t in list(adds):
            adds.pop(t).wait()
        plsc.subcore_barrier()

        def ocopy(j, carry):
            r0 = sid * rows_tile + j * stage
            pltpu.sync_copy(acc_sh.at[pl.ds(r0, stage)], stage_v)
            pltpu.sync_copy(stage_v, out_hbm.at[cid, pl.ds(r0, stage)])
            return carry

        lax.fori_loop(0, nstage, ocopy, 0)

    return k(msg, dst2d)


def _silu(t):
    return t / (1.0 + jnp.exp(-t))


def _init_body(ns, at_ref, ne_ref, wmsg_ref, x_ref, y_ref):
    at = at_ref[:, :]
    x = jnp.zeros((at.shape[0], ne_ref.shape[1]), jnp.float32)
    for s in range(ns):
        x = jnp.where(at == s, ne_ref[s:s + 1, :], x)
    x_ref[:, :] = x
    y_ref[:, :] = jnp.dot(x, wmsg_ref[:, :], preferred_element_type=jnp.float32)


def _tc_init(atom_types2d, node_embed, wmsg0):
    n = atom_types2d.shape[0]
    ns, d = node_embed.shape
    grid = n // _BN
    return pl.pallas_call(
        functools.partial(_init_body, ns),
        grid=(grid,),
        in_specs=[
            pl.BlockSpec((_BN, 1), lambda i: (i, 0)),
            pl.BlockSpec((ns, d), lambda i: (0, 0)),
            pl.BlockSpec((d, d), lambda i: (0, 0)),
        ],
        out_specs=[
            pl.BlockSpec((_BN, d), lambda i: (i, 0)),
            pl.BlockSpec((_BN, d), lambda i: (i, 0)),
        ],
        out_shape=[
            jax.ShapeDtypeStruct((n, d), jnp.float32),
            jax.ShapeDtypeStruct((n, d), jnp.float32),
        ],
    )(atom_types2d, node_embed, wmsg0)


def _edge_body(first, nb, vx_ref, vy_ref, vz_ref, g_ref, wrad_ref, wsh_ref,
               wedge_ref, *rest):
    if first:
        (wei_ref, msg_ref, eo_ref) = rest
    else:
        (e_ref, msg_ref, eo_ref) = rest
    n_col = (lax.broadcasted_iota(jnp.int32, (nb, 1), 0).astype(jnp.float32)
             + 1.0) * (np.pi / RC)
    dn = (((0,), (0,)), ((), ()))
    wrad = wrad_ref[:, :]
    wsh = wsh_ref[:, :]
    wedge = wedge_ref[:, :]
    if first:
        wei = wei_ref[:, :]
    vxb = vx_ref[:].reshape(_BE // 128, 128)
    vyb = vy_ref[:].reshape(_BE // 128, 128)
    vzb = vz_ref[:].reshape(_BE // 128, 128)
    for c in range(_BE // 128):
        x = vxb[c:c + 1, :]
        y = vyb[c:c + 1, :]
        z = vzb[c:c + 1, :]
        r = jnp.sqrt(x * x + y * y + z * z) + 1e-6
        inv = 1.0 / r
        u = jnp.clip(r * (1.0 / RC), 0.0, 1.0)
        u2 = u * u
        u6 = u2 * u2 * u2
        fc = 1.0 - 28.0 * u6 + 48.0 * u6 * u - 21.0 * u6 * u2
        rbt = jnp.sin(n_col * r) * (np.sqrt(2.0 / RC) * inv * fc)
        radial_w = lax.dot_general(rbt, wrad, dn,
                                   preferred_element_type=jnp.float32)
        xh = x * inv
        yh = y * inv
        zh = z * inv
        xx = xh * xh
        yy = yh * yh
        zz = zh * zh
        sht = jnp.concatenate([
            jnp.ones_like(xh),
            xh, yh, zh,
            xh * yh, yh * zh, 2.0 * zz - xx - yy, zh * xh, xx - yy,
            yh * (3.0 * xx - yy), xh * yh * zh, yh * (4.0 * zz - xx - yy),
            zh * (2.0 * zz - 3.0 * xx - 3.0 * yy),
            xh * (4.0 * zz - xx - yy),
            zh * (xx - yy), xh * (xx - 3.0 * yy),
        ], axis=0)
        sh_w = lax.dot_general(sht, wsh, dn,
                               preferred_element_type=jnp.float32)
        if first:
            e = lax.dot_general(rbt, wei, dn,
                                preferred_element_type=jnp.float32)
        else:
            e = e_ref[pl.ds(c * 128, 128), :]
        msg = g_ref[pl.ds(c * 128, 128), :] * radial_w * sh_w + e
        msg_ref[pl.ds(c * 128, 128), :] = msg
        t = jnp.dot(msg, wedge, preferred_element_type=jnp.float32)
        eo_ref[pl.ds(c * 128, 128), :] = e + _silu(t)


def _tc_edge(vx, vy, vz, g, wrad, wsh, wedge, e_or_wei, e_rows, first):
    d = g.shape[1]
    nb = wrad.shape[0]
    grid = -(-e_rows // _BE)
    full = lambda a, b: pl.BlockSpec((a, b), lambda i: (0, 0))
    in_specs = [
        pl.BlockSpec((_BE,), lambda i: (i,)),
        pl.BlockSpec((_BE,), lambda i: (i,)),
        pl.BlockSpec((_BE,), lambda i: (i,)),
        pl.BlockSpec((_BE, d), lambda i: (i, 0)),
        full(nb, d),
        full(16, d),
        full(d, d),
    ]
    if first:
        in_specs.append(full(nb, d))
    else:
        in_specs.append(pl.BlockSpec((_BE, d), lambda i: (i, 0)))
    return pl.pallas_call(
        functools.partial(_edge_body, first, nb),
        grid=(grid,),
        in_specs=in_specs,
        out_specs=[
            pl.BlockSpec((_BE, d), lambda i: (i, 0)),
            pl.BlockSpec((_BE, d), lambda i: (i, 0)),
        ],
        out_shape=[
            jax.ShapeDtypeStruct((g.shape[0], d), jnp.float32),
            jax.ShapeDtypeStruct((e_rows, d), jnp.float32),
        ],
    )(vx, vy, vz, g, wrad, wsh, wedge, e_or_wei)


def _upd_body(last, aggp_ref, x_ref, wupd_ref, wmsg_ref, xo_ref, y_ref):
    agg = aggp_ref[0] + aggp_ref[1]
    t = jnp.dot(agg, wupd_ref[:, :], preferred_element_type=jnp.float32)
    xo = x_ref[:, :] + _silu(t)
    xo_ref[:, :] = xo
    if not last:
        y_ref[:, :] = jnp.dot(xo, wmsg_ref[:, :],
                              preferred_element_type=jnp.float32)


def _tc_update(aggp, x, wupd, wmsg_next, last):
    n, d = x.shape
    grid = n // _BN
    return pl.pallas_call(
        functools.partial(_upd_body, last),
        grid=(grid,),
        in_specs=[
            pl.BlockSpec((_NC, _BN, d), lambda i: (0, i, 0)),
            pl.BlockSpec((_BN, d), lambda i: (i, 0)),
            pl.BlockSpec((d, d), lambda i: (0, 0)),
            pl.BlockSpec((d, d), lambda i: (0, 0)),
        ],
        out_specs=[
            pl.BlockSpec((_BN, d), lambda i: (i, 0)),
            pl.BlockSpec((_BN, d), lambda i: (i, 0)),
        ],
        out_shape=[
            jax.ShapeDtypeStruct((n, d), jnp.float32),
            jax.ShapeDtypeStruct((n, d), jnp.float32),
        ],
    )(aggp, x, wupd, wmsg_next)


def kernel(pos, edge_index, atom_types, node_embed, W_edge_init, W_rad,
           W_sh, W_msg, W_upd, W_edge):
    n, _ = pos.shape
    e = edge_index.shape[1]
    nl = W_rad.shape[0]
    d = node_embed.shape[1]
    npad = -(-n // 1280) * 1280

    src = edge_index[0]
    dst = edge_index[1]
    px = jnp.asarray(pos[:, 0])
    py = jnp.asarray(pos[:, 1])
    pz = jnp.asarray(pos[:, 2])
    at2d = atom_types.reshape(n, 1)

    nchunk = e // _CH
    ncp = -(-nchunk // _NW) * _NW
    src2d = jnp.pad(src.reshape(nchunk, _CH), ((0, ncp - nchunk), (0, 0)))
    dst2d = jnp.pad(dst.reshape(nchunk, _CH), ((0, ncp - nchunk), (0, 0)),
                    constant_values=npad - 1)
    epad = ncp * _CH
    srcp = jnp.pad(src, (0, epad - e))
    dstp = jnp.pad(dst, (0, epad - e))

    vx, vy, vz = _sc_edge_vec(px, py, pz, srcp, dstp)
    x, y = _tc_init(at2d, node_embed, W_msg[0])

    e_cur = None
    for l in range(nl):
        g = _sc_gather_rows(y, src2d)
        if l == 0:
            msg, e_cur = _tc_edge(vx, vy, vz, g, W_rad[0], W_sh[0],
                                  W_edge[0], W_edge_init, e, first=True)
        else:
            msg, e_cur = _tc_edge(vx, vy, vz, g, W_rad[l], W_sh[l],
                                  W_edge[l], e_cur, e, first=False)
        aggp = _sc_scatter_add(msg, dst2d, npad, e)
        wmsg_next = W_msg[l + 1] if l + 1 < nl else W_msg[l]
        x, y = _tc_update(aggp, x, W_upd[l], wmsg_next,
                          last=(l + 1 == nl))
    return (x, e_cur)


# speedup vs baseline: 1.4686x; 1.2439x over previous
import functools

import jax
import jax.numpy as jnp
import numpy as np
from jax import lax
from jax.experimental import pallas as pl
from jax.experimental.pallas import tpu as pltpu
from jax.experimental.pallas import tpu_sc as plsc

RC = 5.0

_NC = 2
_NSC = 16
_NW = _NC * _NSC
_CH = 128

_BE = 2048
_BN = 2000


def _mesh():
    return plsc.VectorSubcoreMesh(
        core_axis_name="c", subcore_axis_name="s",
        num_cores=_NC, num_subcores=_NSC)


_CHV = 1024


def _sc_edge_vec(px, py, pz, srcp, dstp):
    n = px.shape[0]
    epad = srcp.shape[0]
    per = epad // _NW

    @functools.partial(
        pl.kernel,
        out_type=tuple(jax.ShapeDtypeStruct((epad,), jnp.float32)
                       for _ in range(3)),
        mesh=_mesh(),
        scratch_types=[
            pltpu.VMEM((n,), jnp.float32),
            pltpu.VMEM((n,), jnp.float32),
            pltpu.VMEM((n,), jnp.float32),
            pltpu.VMEM((per,), jnp.int32),
            pltpu.VMEM((per,), jnp.int32),
            pltpu.VMEM((per,), jnp.float32),
            pltpu.VMEM((per,), jnp.float32),
            pltpu.VMEM((per,), jnp.float32),
        ],
        compiler_params=pltpu.CompilerParams(needs_layout_passes=False),
    )
    def k(px_hbm, py_hbm, pz_hbm, src_hbm, dst_hbm, vx_hbm, vy_hbm, vz_hbm,
          px_v, py_v, pz_v, si_v, di_v, vx_v, vy_v, vz_v):
        wid = lax.axis_index("s") * _NC + lax.axis_index("c")
        c0 = pl.multiple_of(wid * per, per)
        pltpu.sync_copy(src_hbm.at[pl.ds(c0, per)], si_v)
        pltpu.sync_copy(dst_hbm.at[pl.ds(c0, per)], di_v)
        pltpu.sync_copy(px_hbm, px_v)
        pltpu.sync_copy(py_hbm, py_v)
        pltpu.sync_copy(pz_hbm, pz_v)

        def jstep(j, c):
            o = j * 16
            s16 = si_v[pl.ds(o, 16)]
            d16 = di_v[pl.ds(o, 16)]
            vx_v[pl.ds(o, 16)] = (plsc.load_gather(px_v, [d16]) -
                                  plsc.load_gather(px_v, [s16]))
            vy_v[pl.ds(o, 16)] = (plsc.load_gather(py_v, [d16]) -
                                  plsc.load_gather(py_v, [s16]))
            vz_v[pl.ds(o, 16)] = (plsc.load_gather(pz_v, [d16]) -
                                  plsc.load_gather(pz_v, [s16]))
            return c

        lax.fori_loop(0, per // 16, jstep, 0)
        pltpu.sync_copy(vx_v, vx_hbm.at[pl.ds(c0, per)])
        pltpu.sync_copy(vy_v, vy_hbm.at[pl.ds(c0, per)])
        pltpu.sync_copy(vz_v, vz_hbm.at[pl.ds(c0, per)])

    return k(px, py, pz, srcp, dstp)


def _sc_gather_rows(table, idx2d):
    nchunk_pad = idx2d.shape[0]
    d = table.shape[1]
    per = nchunk_pad // _NW
    kk = 3
    ngroup = -(-per // kk)

    @functools.partial(
        pl.kernel,
        out_type=jax.ShapeDtypeStruct((nchunk_pad * _CH, d), jnp.float32),
        mesh=_mesh(),
        scratch_types=[
            pltpu.VMEM((per, _CH), jnp.int32),
            pltpu.VMEM((2, kk, _CH, d), jnp.float32),
            pltpu.SemaphoreType.DMA,
            pltpu.SemaphoreType.DMA,
            pltpu.SemaphoreType.DMA,
            pltpu.SemaphoreType.DMA,
        ],
    )
    def k(tab_hbm, idx_hbm, out_hbm, idx_v, rows_v, sg0, sg1, sw0, sw1):
        wid = lax.axis_index("s") * _NC + lax.axis_index("c")
        c0 = wid * per
        pltpu.sync_copy(idx_hbm.at[pl.ds(c0, per)], idx_v)
        sg = (sg0, sg1)
        sw = (sw0, sw1)
        gat = {}
        wbk = {}
        for g in range(ngroup):
            bank = g % 2
            if g >= 2:
                for dsc in wbk.pop(g - 2):
                    dsc.wait()
            fired = []
            for j in range(kk):
                t = g * kk + j
                if t >= per:
                    break
                dsc = pltpu.async_copy(
                    tab_hbm.at[idx_v.at[t]], rows_v.at[bank, j], sg[bank])
                fired.append((t, dsc))
            gat[g] = fired
            if g >= 1:
                pb = (g - 1) % 2
                wfired = []
                for (t, dsc) in gat.pop(g - 1):
                    dsc.wait()
                    wdsc = pltpu.async_copy(
                        rows_v.at[pb, t - (g - 1) * kk],
                        out_hbm.at[pl.ds((c0 + t) * _CH, _CH)], sw[pb])
                    wfired.append(wdsc)
                wbk[g - 1] = wfired
        g = ngroup
        pb = (g - 1) % 2
        wfired = []
        for (t, dsc) in gat.pop(g - 1, []):
            dsc.wait()
            wdsc = pltpu.async_copy(
                rows_v.at[pb, t - (g - 1) * kk],
                out_hbm.at[pl.ds((c0 + t) * _CH, _CH)], sw[pb])
            wfired.append(wdsc)
        wbk[g - 1] = wfired
        for fired in wbk.values():
            for dsc in fired:
                dsc.wait()

    return k(table, idx2d)


def _sc_scatter_add(msg, dst2d, npad, evalid):
    e, d = msg.shape
    nchunk_pad = dst2d.shape[0]
    per = nchunk_pad // _NW
    rows_tile = npad // _NSC
    stage = 32
    nstage = rows_tile // stage

    @functools.partial(
        pl.kernel,
        out_type=jax.ShapeDtypeStruct((_NC, npad, d), jnp.float32),
        mesh=_mesh(),
        scratch_types=[
            pltpu.VMEM((per, _CH), jnp.int32),
            pltpu.VMEM((2, _CH, d), jnp.float32),
            pltpu.VMEM((stage, d), jnp.float32),
            pltpu.VMEM_SHARED((npad, d), jnp.float32),
            pltpu.SemaphoreType.DMA,
            pltpu.SemaphoreType.DMA,
            pltpu.SemaphoreType.DMA,
            pltpu.SemaphoreType.DMA,
        ],
    )
    def k(msg_hbm, idx_hbm, out_hbm, idx_v, rows_v, stage_v, acc_sh,
          sm0, sm1, ss0, ss1):
        cid = lax.axis_index("c")
        sid = lax.axis_index("s")
        wid = sid * _NC + cid
        c0 = wid * per
        ipre = pltpu.async_copy(idx_hbm.at[pl.ds(c0, per)], idx_v, sm1)

        def zbody(i, carry):
            for q in range(d // 16):
                stage_v[i, pl.ds(q * 16, 16)] = jnp.zeros((16,), jnp.float32)
            return carry

        lax.fori_loop(0, stage, zbody, 0)

        def zcopy(j, carry):
            pltpu.sync_copy(
                stage_v, acc_sh.at[pl.ds(sid * rows_tile + j * stage, stage)])
            return carry

        lax.fori_loop(0, nstage, zcopy, 0)
        ipre.wait()
        plsc.subcore_barrier()

        sm = (sm0, sm1)
        ss = (ss0, ss1)
        loads = {}
        adds = {}
        for t in range(per):
            b = t % 2
            if t >= 2:
                adds.pop(t - 2).wait()
            base = jnp.minimum((c0 + t) * _CH, evalid - _CH)
            loads[t] = pltpu.async_copy(
                msg_hbm.at[pl.ds(base, _CH)], rows_v.at[b], sm[b])
            if t >= 1:
                loads.pop(t - 1).wait()
                adds[t - 1] = pltpu.async_copy(
                    rows_v.at[(t - 1) % 2], acc_sh.at[idx_v.at[t - 1]],
                    ss[(t - 1) % 2], add=True)
        loads.pop(per - 1).wait()
        adds[per - 1] = pltpu.async_copy(
            rows_v.at[(per - 1) % 2], acc_sh.at[idx_v.at[per - 1]],
            ss[(per - 1) % 2], add=True)
        for t in list(adds):
            adds.pop(t).wait()
        plsc.subcore_barrier()

        def ocopy(j, carry):
            r0 = sid * rows_tile + j * stage
            pltpu.sync_copy(acc_sh.at[pl.ds(r0, stage)], stage_v)
            pltpu.sync_copy(stage_v, out_hbm.at[cid, pl.ds(r0, stage)])
            return carry

        lax.fori_loop(0, nstage, ocopy, 0)

    return k(msg, dst2d)


def _silu(t):
    return t / (1.0 + jnp.exp(-t))


def _init_body(ns, at_ref, ne_ref, wmsg_ref, x_ref, y_ref):
    at = at_ref[:, :]
    x = jnp.zeros((at.shape[0], ne_ref.shape[1]), jnp.float32)
    for s in range(ns):
        x = jnp.where(at == s, ne_ref[s:s + 1, :], x)
    x_ref[:, :] = x
    y_ref[:, :] = jnp.dot(x, wmsg_ref[:, :], preferred_element_type=jnp.float32)


def _tc_init(atom_types2d, node_embed, wmsg0):
    n = atom_types2d.shape[0]
    ns, d = node_embed.shape
    grid = n // _BN
    return pl.pallas_call(
        functools.partial(_init_body, ns),
        grid=(grid,),
        in_specs=[
            pl.BlockSpec((_BN, 1), lambda i: (i, 0)),
            pl.BlockSpec((ns, d), lambda i: (0, 0)),
            pl.BlockSpec((d, d), lambda i: (0, 0)),
        ],
        out_specs=[
            pl.BlockSpec((_BN, d), lambda i: (i, 0)),
            pl.BlockSpec((_BN, d), lambda i: (i, 0)),
        ],
        out_shape=[
            jax.ShapeDtypeStruct((n, d), jnp.float32),
            jax.ShapeDtypeStruct((n, d), jnp.float32),
        ],
    )(atom_types2d, node_embed, wmsg0)


def _edge_body(first, nb, vx_ref, vy_ref, vz_ref, g_ref, wrad_ref, wsh_ref,
               wedge_ref, *rest):
    if first:
        (wei_ref, msg_ref, eo_ref) = rest
    else:
        (e_ref, msg_ref, eo_ref) = rest
    dn = (((0,), (0,)), ((), ()))
    wrad = wrad_ref[:, :]
    wsh = wsh_ref[:, :]
    wedge = wedge_ref[:, :]
    if first:
        wei = wei_ref[:, :]
    vxb = vx_ref[:].reshape(_BE // 128, 128)
    vyb = vy_ref[:].reshape(_BE // 128, 128)
    vzb = vz_ref[:].reshape(_BE // 128, 128)
    for c in range(_BE // 128):
        x = vxb[c:c + 1, :]
        y = vyb[c:c + 1, :]
        z = vzb[c:c + 1, :]
        r = jnp.sqrt(x * x + y * y + z * z) + 1e-6
        inv = 1.0 / r
        u = jnp.clip(r * (1.0 / RC), 0.0, 1.0)
        u2 = u * u
        u6 = u2 * u2 * u2
        fc = 1.0 - 28.0 * u6 + 48.0 * u6 * u - 21.0 * u6 * u2
        theta = r * (np.pi / RC)
        sb = jnp.sin(theta)
        cb = jnp.cos(theta)
        while sb.shape[0] < nb:
            kr = sb.shape[0]
            sk = sb[kr - 1:kr, :]
            ck = cb[kr - 1:kr, :]
            sb = jnp.concatenate([sb, sb * ck + cb * sk], axis=0)
            cb = jnp.concatenate([cb, cb * ck - sb[:kr] * sk], axis=0)
        rbt = sb * (np.sqrt(2.0 / RC) * inv * fc)
        radial_w = lax.dot_general(rbt, wrad, dn,
                                   preferred_element_type=jnp.float32)
        xh = x * inv
        yh = y * inv
        zh = z * inv
        xx = xh * xh
        yy = yh * yh
        zz = zh * zh
        sht = jnp.concatenate([
            jnp.ones_like(xh),
            xh, yh, zh,
            xh * yh, yh * zh, 2.0 * zz - xx - yy, zh * xh, xx - yy,
            yh * (3.0 * xx - yy), xh * yh * zh, yh * (4.0 * zz - xx - yy),
            zh * (2.0 * zz - 3.0 * xx - 3.0 * yy),
            xh * (4.0 * zz - xx - yy),
            zh * (xx - yy), xh * (xx - 3.0 * yy),
        ], axis=0)
        sh_w = lax.dot_general(sht, wsh, dn,
                               preferred_element_type=jnp.float32)
        if first:
            e = lax.dot_general(rbt, wei, dn,
                                preferred_element_type=jnp.float32)
        else:
            e = e_ref[pl.ds(c * 128, 128), :]
        msg = g_ref[pl.ds(c * 128, 128), :] * radial_w * sh_w + e
        msg_ref[pl.ds(c * 128, 128), :] = msg
        t = jnp.dot(msg, wedge, preferred_element_type=jnp.float32)
        eo_ref[pl.ds(c * 128, 128), :] = e + _silu(t)


def _tc_edge(vx, vy, vz, g, wrad, wsh, wedge, e_or_wei, e_rows, first):
    d = g.shape[1]
    nb = wrad.shape[0]
    grid = -(-e_rows // _BE)
    full = lambda a, b: pl.BlockSpec((a, b), lambda i: (0, 0))
    in_specs = [
        pl.BlockSpec((_BE,), lambda i: (i,)),
        pl.BlockSpec((_BE,), lambda i: (i,)),
        pl.BlockSpec((_BE,), lambda i: (i,)),
        pl.BlockSpec((_BE, d), lambda i: (i, 0)),
        full(nb, d),
        full(16, d),
        full(d, d),
    ]
    if first:
        in_specs.append(full(nb, d))
    else:
        in_specs.append(pl.BlockSpec((_BE, d), lambda i: (i, 0)))
    return pl.pallas_call(
        functools.partial(_edge_body, first, nb),
        grid=(grid,),
        in_specs=in_specs,
        out_specs=[
            pl.BlockSpec((_BE, d), lambda i: (i, 0)),
            pl.BlockSpec((_BE, d), lambda i: (i, 0)),
        ],
        out_shape=[
            jax.ShapeDtypeStruct((g.shape[0], d), jnp.float32),
            jax.ShapeDtypeStruct((e_rows, d), jnp.float32),
        ],
    )(vx, vy, vz, g, wrad, wsh, wedge, e_or_wei)


def _upd_body(last, aggp_ref, x_ref, wupd_ref, wmsg_ref, xo_ref, y_ref):
    agg = aggp_ref[0] + aggp_ref[1]
    t = jnp.dot(agg, wupd_ref[:, :], preferred_element_type=jnp.float32)
    xo = x_ref[:, :] + _silu(t)
    xo_ref[:, :] = xo
    if not last:
        y_ref[:, :] = jnp.dot(xo, wmsg_ref[:, :],
                              preferred_element_type=jnp.float32)


def _tc_update(aggp, x, wupd, wmsg_next, last):
    n, d = x.shape
    grid = n // _BN
    return pl.pallas_call(
        functools.partial(_upd_body, last),
        grid=(grid,),
        in_specs=[
            pl.BlockSpec((_NC, _BN, d), lambda i: (0, i, 0)),
            pl.BlockSpec((_BN, d), lambda i: (i, 0)),
            pl.BlockSpec((d, d), lambda i: (0, 0)),
            pl.BlockSpec((d, d), lambda i: (0, 0)),
        ],
        out_specs=[
            pl.BlockSpec((_BN, d), lambda i: (i, 0)),
            pl.BlockSpec((_BN, d), lambda i: (i, 0)),
        ],
        out_shape=[
            jax.ShapeDtypeStruct((n, d), jnp.float32),
            jax.ShapeDtypeStruct((n, d), jnp.float32),
        ],
    )(aggp, x, wupd, wmsg_next)


def kernel(pos, edge_index, atom_types, node_embed, W_edge_init, W_rad,
           W_sh, W_msg, W_upd, W_edge):
    n, _ = pos.shape
    e = edge_index.shape[1]
    nl = W_rad.shape[0]
    d = node_embed.shape[1]
    npad = -(-n // 1280) * 1280

    src = edge_index[0]
    dst = edge_index[1]
    px = jnp.asarray(pos[:, 0])
    py = jnp.asarray(pos[:, 1])
    pz = jnp.asarray(pos[:, 2])
    at2d = atom_types.reshape(n, 1)

    nchunk = e // _CH
    ncp = -(-nchunk // _NW) * _NW
    src2d = jnp.pad(src.reshape(nchunk, _CH), ((0, ncp - nchunk), (0, 0)))
    dst2d = jnp.pad(dst.reshape(nchunk, _CH), ((0, ncp - nchunk), (0, 0)),
                    constant_values=npad - 1)
    epad = ncp * _CH
    srcp = jnp.pad(src, (0, epad - e))
    dstp = jnp.pad(dst, (0, epad - e))

    vx, vy, vz = _sc_edge_vec(px, py, pz, srcp, dstp)
    x, y = _tc_init(at2d, node_embed, W_msg[0])

    e_cur = None
    for l in range(nl):
        g = _sc_gather_rows(y, src2d)
        if l == 0:
            msg, e_cur = _tc_edge(vx, vy, vz, g, W_rad[0], W_sh[0],
                                  W_edge[0], W_edge_init, e, first=True)
        else:
            msg, e_cur = _tc_edge(vx, vy, vz, g, W_rad[l], W_sh[l],
                                  W_edge[l], e_cur, e, first=False)
        aggp = _sc_scatter_add(msg, dst2d, npad, e)
        wmsg_next = W_msg[l + 1] if l + 1 < nl else W_msg[l]
        x, y = _tc_update(aggp, x, W_upd[l], wmsg_next,
                          last=(l + 1 == nl))
    return (x, e_cur)


# speedup vs baseline: 2.5081x; 1.7078x over previous
import functools

import jax
import jax.numpy as jnp
import numpy as np
from jax import lax
from jax.experimental import pallas as pl
from jax.experimental.pallas import tpu as pltpu
from jax.experimental.pallas import tpu_sc as plsc

RC = 5.0

_NC = 2
_NSC = 16
_NW = _NC * _NSC
_CH = 128

_BE = 2048
_BN = 2000


def _mesh():
    return plsc.VectorSubcoreMesh(
        core_axis_name="c", subcore_axis_name="s",
        num_cores=_NC, num_subcores=_NSC)


_CHV = 1024


def _sc_edge_vec(px, py, pz, srcp, dstp):
    n = px.shape[0]
    epad = srcp.shape[0]
    per = epad // _NW

    @functools.partial(
        pl.kernel,
        out_type=tuple(jax.ShapeDtypeStruct((epad,), jnp.float32)
                       for _ in range(3)),
        mesh=_mesh(),
        scratch_types=[
            pltpu.VMEM((n,), jnp.float32),
            pltpu.VMEM((n,), jnp.float32),
            pltpu.VMEM((n,), jnp.float32),
            pltpu.VMEM((per,), jnp.int32),
            pltpu.VMEM((per,), jnp.int32),
            pltpu.VMEM((per,), jnp.float32),
            pltpu.VMEM((per,), jnp.float32),
            pltpu.VMEM((per,), jnp.float32),
        ],
        compiler_params=pltpu.CompilerParams(needs_layout_passes=False),
    )
    def k(px_hbm, py_hbm, pz_hbm, src_hbm, dst_hbm, vx_hbm, vy_hbm, vz_hbm,
          px_v, py_v, pz_v, si_v, di_v, vx_v, vy_v, vz_v):
        wid = lax.axis_index("s") * _NC + lax.axis_index("c")
        c0 = pl.multiple_of(wid * per, per)
        pltpu.sync_copy(src_hbm.at[pl.ds(c0, per)], si_v)
        pltpu.sync_copy(dst_hbm.at[pl.ds(c0, per)], di_v)
        pltpu.sync_copy(px_hbm, px_v)
        pltpu.sync_copy(py_hbm, py_v)
        pltpu.sync_copy(pz_hbm, pz_v)

        def jstep(j, c):
            o = j * 16
            s16 = si_v[pl.ds(o, 16)]
            d16 = di_v[pl.ds(o, 16)]
            vx_v[pl.ds(o, 16)] = (plsc.load_gather(px_v, [d16]) -
                                  plsc.load_gather(px_v, [s16]))
            vy_v[pl.ds(o, 16)] = (plsc.load_gather(py_v, [d16]) -
                                  plsc.load_gather(py_v, [s16]))
            vz_v[pl.ds(o, 16)] = (plsc.load_gather(pz_v, [d16]) -
                                  plsc.load_gather(pz_v, [s16]))
            return c

        lax.fori_loop(0, per // 16, jstep, 0)
        pltpu.sync_copy(vx_v, vx_hbm.at[pl.ds(c0, per)])
        pltpu.sync_copy(vy_v, vy_hbm.at[pl.ds(c0, per)])
        pltpu.sync_copy(vz_v, vz_hbm.at[pl.ds(c0, per)])

    return k(px, py, pz, srcp, dstp)


def _sc_gather_rows(table, idx2d):
    nchunk_pad = idx2d.shape[0]
    ntab, d = table.shape
    per = nchunk_pad // _NW
    b8 = (ntab // _NSC) // 8 * 8
    rem = ntab - b8 * _NSC

    @functools.partial(
        pl.kernel,
        out_type=jax.ShapeDtypeStruct((nchunk_pad * _CH, d), jnp.float32),
        mesh=_mesh(),
        scratch_types=[
            pltpu.VMEM((per, _CH), jnp.int32),
            pltpu.VMEM((2, _CH, d), jnp.float32),
            pltpu.VMEM_SHARED((ntab, d), jnp.float32),
            pltpu.SemaphoreType.DMA,
            pltpu.SemaphoreType.DMA,
            pltpu.SemaphoreType.DMA,
            pltpu.SemaphoreType.DMA,
        ],
    )
    def k(tab_hbm, idx_hbm, out_hbm, idx_v, rows_v, tab_sh, sg0, sg1,
          sw0, sw1):
        cid = lax.axis_index("c")
        sid = lax.axis_index("s")
        wid = sid * _NC + cid
        c0 = wid * per
        ipre = pltpu.async_copy(idx_hbm.at[pl.ds(c0, per)], idx_v, sw0)
        r0 = pl.multiple_of(sid * b8, 8)
        pltpu.sync_copy(tab_hbm.at[pl.ds(r0, b8)], tab_sh.at[pl.ds(r0, b8)])
        if rem:
            @pl.when(sid == 0)
            def _():
                pltpu.sync_copy(tab_hbm.at[pl.ds(b8 * _NSC, rem)],
                                tab_sh.at[pl.ds(b8 * _NSC, rem)])
        ipre.wait()
        plsc.subcore_barrier()

        sg = (sg0, sg1)
        sw = (sw0, sw1)
        gat = {}
        wbk = {}
        for t in range(per):
            b = t % 2
            if t >= 2:
                wbk.pop(t - 2).wait()
            gat[t] = pltpu.async_copy(
                tab_sh.at[idx_v.at[t]], rows_v.at[b], sg[b])
            if t >= 1:
                gat.pop(t - 1).wait()
                wbk[t - 1] = pltpu.async_copy(
                    rows_v.at[(t - 1) % 2],
                    out_hbm.at[pl.ds((c0 + t - 1) * _CH, _CH)], sw[(t - 1) % 2])
        gat.pop(per - 1).wait()
        wbk[per - 1] = pltpu.async_copy(
            rows_v.at[(per - 1) % 2],
            out_hbm.at[pl.ds((c0 + per - 1) * _CH, _CH)], sw[(per - 1) % 2])
        for t in list(wbk):
            wbk.pop(t).wait()

    return k(table, idx2d)


def _sc_scatter_add(msg, dst2d, npad, evalid):
    e, d = msg.shape
    nchunk_pad = dst2d.shape[0]
    per = nchunk_pad // _NW
    rows_tile = npad // _NSC
    stage = 32
    nstage = rows_tile // stage

    @functools.partial(
        pl.kernel,
        out_type=jax.ShapeDtypeStruct((_NC, npad, d), jnp.float32),
        mesh=_mesh(),
        scratch_types=[
            pltpu.VMEM((per, _CH), jnp.int32),
            pltpu.VMEM((2, _CH, d), jnp.float32),
            pltpu.VMEM((stage, d), jnp.float32),
            pltpu.VMEM_SHARED((npad, d), jnp.float32),
            pltpu.SemaphoreType.DMA,
            pltpu.SemaphoreType.DMA,
            pltpu.SemaphoreType.DMA,
            pltpu.SemaphoreType.DMA,
        ],
    )
    def k(msg_hbm, idx_hbm, out_hbm, idx_v, rows_v, stage_v, acc_sh,
          sm0, sm1, ss0, ss1):
        cid = lax.axis_index("c")
        sid = lax.axis_index("s")
        wid = sid * _NC + cid
        c0 = wid * per
        ipre = pltpu.async_copy(idx_hbm.at[pl.ds(c0, per)], idx_v, sm1)

        def zbody(i, carry):
            for q in range(d // 16):
                stage_v[i, pl.ds(q * 16, 16)] = jnp.zeros((16,), jnp.float32)
            return carry

        lax.fori_loop(0, stage, zbody, 0)

        def zcopy(j, carry):
            pltpu.sync_copy(
                stage_v, acc_sh.at[pl.ds(sid * rows_tile + j * stage, stage)])
            return carry

        lax.fori_loop(0, nstage, zcopy, 0)
        ipre.wait()
        plsc.subcore_barrier()

        sm = (sm0, sm1)
        ss = (ss0, ss1)
        loads = {}
        adds = {}
        for t in range(per):
            b = t % 2
            if t >= 2:
                adds.pop(t - 2).wait()
            base = jnp.minimum((c0 + t) * _CH, evalid - _CH)
            loads[t] = pltpu.async_copy(
                msg_hbm.at[pl.ds(base, _CH)], rows_v.at[b], sm[b])
            if t >= 1:
                loads.pop(t - 1).wait()
                adds[t - 1] = pltpu.async_copy(
                    rows_v.at[(t - 1) % 2], acc_sh.at[idx_v.at[t - 1]],
                    ss[(t - 1) % 2], add=True)
        loads.pop(per - 1).wait()
        adds[per - 1] = pltpu.async_copy(
            rows_v.at[(per - 1) % 2], acc_sh.at[idx_v.at[per - 1]],
            ss[(per - 1) % 2], add=True)
        for t in list(adds):
            adds.pop(t).wait()
        plsc.subcore_barrier()

        def ocopy(j, carry):
            r0 = sid * rows_tile + j * stage
            pltpu.sync_copy(acc_sh.at[pl.ds(r0, stage)], stage_v)
            pltpu.sync_copy(stage_v, out_hbm.at[cid, pl.ds(r0, stage)])
            return carry

        lax.fori_loop(0, nstage, ocopy, 0)

    return k(msg, dst2d)


def _silu(t):
    return t / (1.0 + jnp.exp(-t))


def _init_body(ns, at_ref, ne_ref, wmsg_ref, x_ref, y_ref):
    at = at_ref[:, :]
    x = jnp.zeros((at.shape[0], ne_ref.shape[1]), jnp.float32)
    for s in range(ns):
        x = jnp.where(at == s, ne_ref[s:s + 1, :], x)
    x_ref[:, :] = x
    y_ref[:, :] = jnp.dot(x, wmsg_ref[:, :], preferred_element_type=jnp.float32)


def _tc_init(atom_types2d, node_embed, wmsg0):
    n = atom_types2d.shape[0]
    ns, d = node_embed.shape
    grid = n // _BN
    return pl.pallas_call(
        functools.partial(_init_body, ns),
        grid=(grid,),
        in_specs=[
            pl.BlockSpec((_BN, 1), lambda i: (i, 0)),
            pl.BlockSpec((ns, d), lambda i: (0, 0)),
            pl.BlockSpec((d, d), lambda i: (0, 0)),
        ],
        out_specs=[
            pl.BlockSpec((_BN, d), lambda i: (i, 0)),
            pl.BlockSpec((_BN, d), lambda i: (i, 0)),
        ],
        out_shape=[
            jax.ShapeDtypeStruct((n, d), jnp.float32),
            jax.ShapeDtypeStruct((n, d), jnp.float32),
        ],
    )(atom_types2d, node_embed, wmsg0)


def _edge_body(first, nb, vx_ref, vy_ref, vz_ref, g_ref, wrad_ref, wsh_ref,
               wedge_ref, *rest):
    if first:
        (wei_ref, msg_ref, eo_ref) = rest
    else:
        (e_ref, msg_ref, eo_ref) = rest
    dn = (((0,), (0,)), ((), ()))
    wrad = wrad_ref[:, :]
    wsh = wsh_ref[:, :]
    wedge = wedge_ref[:, :]
    if first:
        wei = wei_ref[:, :]
    vxb = vx_ref[:].reshape(_BE // 128, 128)
    vyb = vy_ref[:].reshape(_BE // 128, 128)
    vzb = vz_ref[:].reshape(_BE // 128, 128)
    for c in range(_BE // 128):
        x = vxb[c:c + 1, :]
        y = vyb[c:c + 1, :]
        z = vzb[c:c + 1, :]
        r = jnp.sqrt(x * x + y * y + z * z) + 1e-6
        inv = 1.0 / r
        u = jnp.clip(r * (1.0 / RC), 0.0, 1.0)
        u2 = u * u
        u6 = u2 * u2 * u2
        fc = 1.0 - 28.0 * u6 + 48.0 * u6 * u - 21.0 * u6 * u2
        theta = r * (np.pi / RC)
        sb = jnp.sin(theta)
        cb = jnp.cos(theta)
        while sb.shape[0] < nb:
            kr = sb.shape[0]
            sk = sb[kr - 1:kr, :]
            ck = cb[kr - 1:kr, :]
            sb = jnp.concatenate([sb, sb * ck + cb * sk], axis=0)
            cb = jnp.concatenate([cb, cb * ck - sb[:kr] * sk], axis=0)
        rbt = sb * (np.sqrt(2.0 / RC) * inv * fc)
        radial_w = lax.dot_general(rbt, wrad, dn,
                                   preferred_element_type=jnp.float32)
        xh = x * inv
        yh = y * inv
        zh = z * inv
        xx = xh * xh
        yy = yh * yh
        zz = zh * zh
        sht = jnp.concatenate([
            jnp.ones_like(xh),
            xh, yh, zh,
            xh * yh, yh * zh, 2.0 * zz - xx - yy, zh * xh, xx - yy,
            yh * (3.0 * xx - yy), xh * yh * zh, yh * (4.0 * zz - xx - yy),
            zh * (2.0 * zz - 3.0 * xx - 3.0 * yy),
            xh * (4.0 * zz - xx - yy),
            zh * (xx - yy), xh * (xx - 3.0 * yy),
        ], axis=0)
        sh_w = lax.dot_general(sht, wsh, dn,
                               preferred_element_type=jnp.float32)
        if first:
            e = lax.dot_general(rbt, wei, dn,
                                preferred_element_type=jnp.float32)
        else:
            e = e_ref[pl.ds(c * 128, 128), :]
        msg = g_ref[pl.ds(c * 128, 128), :] * radial_w * sh_w + e
        msg_ref[pl.ds(c * 128, 128), :] = msg
        t = jnp.dot(msg, wedge, preferred_element_type=jnp.float32)
        eo_ref[pl.ds(c * 128, 128), :] = e + _silu(t)


def _tc_edge(vx, vy, vz, g, wrad, wsh, wedge, e_or_wei, e_rows, first):
    d = g.shape[1]
    nb = wrad.shape[0]
    grid = -(-e_rows // _BE)
    full = lambda a, b: pl.BlockSpec((a, b), lambda i: (0, 0))
    in_specs = [
        pl.BlockSpec((_BE,), lambda i: (i,)),
        pl.BlockSpec((_BE,), lambda i: (i,)),
        pl.BlockSpec((_BE,), lambda i: (i,)),
        pl.BlockSpec((_BE, d), lambda i: (i, 0)),
        full(nb, d),
        full(16, d),
        full(d, d),
    ]
    if first:
        in_specs.append(full(nb, d))
    else:
        in_specs.append(pl.BlockSpec((_BE, d), lambda i: (i, 0)))
    return pl.pallas_call(
        functools.partial(_edge_body, first, nb),
        grid=(grid,),
        in_specs=in_specs,
        out_specs=[
            pl.BlockSpec((_BE, d), lambda i: (i, 0)),
            pl.BlockSpec((_BE, d), lambda i: (i, 0)),
        ],
        out_shape=[
            jax.ShapeDtypeStruct((g.shape[0], d), jnp.float32),
            jax.ShapeDtypeStruct((e_rows, d), jnp.float32),
        ],
    )(vx, vy, vz, g, wrad, wsh, wedge, e_or_wei)


def _upd_body(last, aggp_ref, x_ref, wupd_ref, wmsg_ref, xo_ref, y_ref):
    agg = aggp_ref[0] + aggp_ref[1]
    t = jnp.dot(agg, wupd_ref[:, :], preferred_element_type=jnp.float32)
    xo = x_ref[:, :] + _silu(t)
    xo_ref[:, :] = xo
    if not last:
        y_ref[:, :] = jnp.dot(xo, wmsg_ref[:, :],
                              preferred_element_type=jnp.float32)


def _tc_update(aggp, x, wupd, wmsg_next, last):
    n, d = x.shape
    grid = n // _BN
    return pl.pallas_call(
        functools.partial(_upd_body, last),
        grid=(grid,),
        in_specs=[
            pl.BlockSpec((_NC, _BN, d), lambda i: (0, i, 0)),
            pl.BlockSpec((_BN, d), lambda i: (i, 0)),
            pl.BlockSpec((d, d), lambda i: (0, 0)),
            pl.BlockSpec((d, d), lambda i: (0, 0)),
        ],
        out_specs=[
            pl.BlockSpec((_BN, d), lambda i: (i, 0)),
            pl.BlockSpec((_BN, d), lambda i: (i, 0)),
        ],
        out_shape=[
            jax.ShapeDtypeStruct((n, d), jnp.float32),
            jax.ShapeDtypeStruct((n, d), jnp.float32),
        ],
    )(aggp, x, wupd, wmsg_next)


def kernel(pos, edge_index, atom_types, node_embed, W_edge_init, W_rad,
           W_sh, W_msg, W_upd, W_edge):
    n, _ = pos.shape
    e = edge_index.shape[1]
    nl = W_rad.shape[0]
    d = node_embed.shape[1]
    npad = -(-n // 1280) * 1280

    src = edge_index[0]
    dst = edge_index[1]
    px = jnp.asarray(pos[:, 0])
    py = jnp.asarray(pos[:, 1])
    pz = jnp.asarray(pos[:, 2])
    at2d = atom_types.reshape(n, 1)

    nchunk = e // _CH
    ncp = -(-nchunk // _NW) * _NW
    src2d = jnp.pad(src.reshape(nchunk, _CH), ((0, ncp - nchunk), (0, 0)))
    dst2d = jnp.pad(dst.reshape(nchunk, _CH), ((0, ncp - nchunk), (0, 0)),
                    constant_values=npad - 1)
    epad = ncp * _CH
    srcp = jnp.pad(src, (0, epad - e))
    dstp = jnp.pad(dst, (0, epad - e))

    vx, vy, vz = _sc_edge_vec(px, py, pz, srcp, dstp)
    x, y = _tc_init(at2d, node_embed, W_msg[0])

    e_cur = None
    for l in range(nl):
        g = _sc_gather_rows(y, src2d)
        if l == 0:
            msg, e_cur = _tc_edge(vx, vy, vz, g, W_rad[0], W_sh[0],
                                  W_edge[0], W_edge_init, e, first=True)
        else:
            msg, e_cur = _tc_edge(vx, vy, vz, g, W_rad[l], W_sh[l],
                                  W_edge[l], e_cur, e, first=False)
        aggp = _sc_scatter_add(msg, dst2d, npad, e)
        wmsg_next = W_msg[l + 1] if l + 1 < nl else W_msg[l]
        x, y = _tc_update(aggp, x, W_upd[l], wmsg_next,
                          last=(l + 1 == nl))
    return (x, e_cur)
